# baseline (speedup 1.0000x reference)
import functools

import jax
import jax.numpy as jnp
from jax import lax
from jax.experimental import pallas as pl
from jax.experimental.pallas import tpu as pltpu
from jax.experimental.pallas import tpu_sc as plsc

SIZE = 1024
LATENT_DIM = 256
BETA_C = 0.25
N_ROWS = 4 * 8 * 32 * 32
TILE = 1024
N_TILES = N_ROWS // TILE
TILES_PER_B = 8192 // TILE

NC, NS, LANES = 2, 16, 16
NW = NC * NS
ROWS_PER_W = N_ROWS // NW
ENC_ELEMS = N_ROWS * SIZE
ZROWS = 64
SCAT = 128
N_SCAT = ROWS_PER_W // SCAT

_sc_mesh = plsc.VectorSubcoreMesh(core_axis_name="c", subcore_axis_name="s")


def _vq_body(zb_ref, emb_ref, zq_ref, idx_ref, loss_ref, perp_ref,
             counts_ref, loss_acc_ref, en_ref):
    t = pl.program_id(0)

    e = emb_ref[...]

    @pl.when(t == 0)
    def _init():
        counts_ref[...] = jnp.zeros_like(counts_ref)
        loss_acc_ref[0] = 0.0
        en_ref[...] = jnp.sum(e * e, axis=1, keepdims=True).T

    zb = zb_ref[0]
    en = en_ref[0]

    ez = jax.lax.dot_general(e, zb, (((1,), (0,)), ((), ())),
                             preferred_element_type=jnp.float32)
    s = en[:, None] - 2.0 * ez

    minv = jnp.min(s, axis=0)
    code_iota = jax.lax.broadcasted_iota(jnp.int32, (SIZE, TILE), 0)
    idx = jnp.min(jnp.where(s == minv[None, :], code_iota, SIZE), axis=0)

    oh_T = (code_iota == idx[None, :]).astype(jnp.float32)
    zqT = jax.lax.dot_general(e, oh_T, (((0,), (0,)), ((), ())),
                              preferred_element_type=jnp.float32)
    zq_ref[0] = zb + (zqT - zb)

    idx_ref[0, 0] = idx

    ones8 = jnp.ones((TILE, 8), jnp.float32)
    counts_ref[...] += jax.lax.dot_general(
        oh_T, ones8, (((1,), (0,)), ((), ())),
        preferred_element_type=jnp.float32)

    zn = jnp.sum(zb * zb, axis=0)
    loss_acc_ref[0] += jnp.sum(zn + minv)

    @pl.when(t == N_TILES - 1)
    def _finish():
        loss_ref[0, 0] = BETA_C * loss_acc_ref[0] / (N_ROWS * LATENT_DIM)
        e_mean = counts_ref[...] / N_ROWS
        ent = jnp.sum(e_mean * jnp.log(e_mean + 1e-10)) / 8.0
        perp_ref[0, 0] = jnp.exp(-ent)


@functools.partial(
    pl.kernel, mesh=_sc_mesh,
    out_type=jax.ShapeDtypeStruct((ENC_ELEMS,), jnp.float32),
    scratch_types=[pltpu.VMEM((ZROWS * SIZE,), jnp.float32)])
def _enc_zero(enc_ref, zbuf):
    w = lax.axis_index("s") * NC + lax.axis_index("c")

    def zb_body(i, c):
        zbuf[pl.ds(i * LANES, LANES)] = jnp.zeros((LANES,), jnp.float32)
        return c
    lax.fori_loop(0, ZROWS * SIZE // LANES, zb_body, 0)

    base = w * ROWS_PER_W * SIZE

    def dma_body(j, c):
        pltpu.sync_copy(zbuf, enc_ref.at[pl.ds(base + j * (ZROWS * SIZE),
                                               ZROWS * SIZE)])
        return c
    lax.fori_loop(0, ROWS_PER_W // ZROWS, dma_body, 0)


@functools.partial(
    pl.kernel, mesh=_sc_mesh,
    scratch_types=[pltpu.VMEM((ROWS_PER_W,), jnp.int32),
                   pltpu.VMEM((N_SCAT, SCAT), jnp.int32),
                   pltpu.VMEM((SCAT,), jnp.float32),
                   pltpu.SemaphoreType.DMA])
def _enc_scatter(idx_hbm, enc_ref, idxv, addrv, onesv, sem):
    w = lax.axis_index("s") * NC + lax.axis_index("c")
    rbase = w * ROWS_PER_W
    pltpu.sync_copy(idx_hbm.at[pl.ds(rbase, ROWS_PER_W)], idxv)

    def ones_body(i, c):
        onesv[pl.ds(i * LANES, LANES)] = jnp.ones((LANES,), jnp.float32)
        return c
    lax.fori_loop(0, SCAT // LANES, ones_body, 0)

    vecs_per_scat = SCAT // LANES

    def addr_body(i, c):
        r0 = i * LANES
        iv = idxv[pl.ds(r0, LANES)]
        lane = lax.iota(jnp.int32, LANES)
        addr = (rbase + r0 + lane) * SIZE + iv
        addrv[i // vecs_per_scat, pl.ds((i % vecs_per_scat) * LANES, LANES)] = addr
        return c
    lax.fori_loop(0, ROWS_PER_W // LANES, addr_body, 0, unroll=vecs_per_scat)

    def scat_body(c, carry):
        pltpu.async_copy(onesv, enc_ref.at[addrv.at[c]], sem).wait()
        return carry
    lax.fori_loop(0, N_SCAT, scat_body, 0)


def kernel(z, embedding_weight):
    z2 = z.reshape(4, 256, 8192)
    zq3, idx3, loss, perp = pl.pallas_call(
        _vq_body,
        grid=(N_TILES,),
        in_specs=[
            pl.BlockSpec((1, 256, TILE), lambda t: (t // TILES_PER_B, 0, t % TILES_PER_B)),
            pl.BlockSpec((SIZE, LATENT_DIM), lambda t: (0, 0)),
        ],
        out_specs=[
            pl.BlockSpec((1, 256, TILE), lambda t: (t // TILES_PER_B, 0, t % TILES_PER_B)),
            pl.BlockSpec((1, 1, TILE), lambda t: (t, 0, 0)),
            pl.BlockSpec((1, 1), lambda t: (0, 0), memory_space=pltpu.SMEM),
            pl.BlockSpec((1, 1), lambda t: (0, 0), memory_space=pltpu.SMEM),
        ],
        out_shape=[
            jax.ShapeDtypeStruct((4, 256, 8192), jnp.float32),
            jax.ShapeDtypeStruct((N_TILES, 1, TILE), jnp.int32),
            jax.ShapeDtypeStruct((1, 1), jnp.float32),
            jax.ShapeDtypeStruct((1, 1), jnp.float32),
        ],
        scratch_shapes=[
            pltpu.VMEM((SIZE, 8), jnp.float32),
            pltpu.SMEM((1,), jnp.float32),
            pltpu.VMEM((1, SIZE), jnp.float32),
        ],
    )(z2, embedding_weight)

    enc_zeroed = _enc_zero()
    enc_ref = jax.new_ref(enc_zeroed)
    _enc_scatter(idx3.reshape(N_ROWS), enc_ref)
    enc = enc_ref[...].reshape(N_ROWS, SIZE)

    z_q_out = zq3.reshape(4, 256, 8, 32, 32)
    min_idx = idx3.reshape(N_ROWS, 1)
    return (z_q_out, loss[0, 0], perp[0, 0], enc, min_idx)


# speedup vs baseline: 2.7256x; 2.7256x over previous
import jax
import jax.numpy as jnp
from jax.experimental import pallas as pl
from jax.experimental.pallas import tpu as pltpu

SIZE = 1024
LATENT_DIM = 256
BETA_C = 0.25
N_ROWS = 4 * 8 * 32 * 32
TILE = 1024
N_TILES = N_ROWS // TILE


def _vq_body(zr_ref, emb_ref, zq_ref, enc_ref, idx_ref, loss_ref, perp_ref,
             counts_ref, loss_acc_ref, en_ref):
    t = pl.program_id(0)

    e = emb_ref[...]

    @pl.when(t == 0)
    def _init():
        counts_ref[...] = jnp.zeros_like(counts_ref)
        loss_acc_ref[0] = 0.0
        en_ref[...] = jnp.sum(e * e, axis=1, keepdims=True).T

    zr = zr_ref[...]
    en = en_ref[...]

    ze = jax.lax.dot_general(zr, e, (((1,), (1,)), ((), ())),
                             preferred_element_type=jnp.float32)
    s = en - 2.0 * ze

    minv = jnp.min(s, axis=1)
    code_iota = jax.lax.broadcasted_iota(jnp.int32, (TILE, SIZE), 1)
    idx = jnp.min(jnp.where(s == minv[:, None], code_iota, SIZE), axis=1)

    oh = (code_iota == idx[:, None]).astype(jnp.float32)
    enc_ref[...] = oh
    zq = jax.lax.dot_general(oh, e, (((1,), (0,)), ((), ())),
                             preferred_element_type=jnp.float32)
    zq_ref[...] = zr + (zq - zr)

    counts_ref[...] += jnp.sum(oh, axis=0, keepdims=True)
    idx_ref[0, 0] = idx

    zn = jnp.sum(zr * zr, axis=1)
    loss_acc_ref[0] += jnp.sum(zn + minv)

    @pl.when(t == N_TILES - 1)
    def _finish():
        loss_ref[0, 0] = BETA_C * loss_acc_ref[0] / (N_ROWS * LATENT_DIM)
        e_mean = counts_ref[...] / N_ROWS
        perp_ref[0, 0] = jnp.exp(-jnp.sum(e_mean * jnp.log(e_mean + 1e-10)))


def kernel(z, embedding_weight):
    zp = jnp.transpose(z, (0, 2, 3, 4, 1)).reshape(N_ROWS, LATENT_DIM)
    zq, enc, idx3, loss, perp = pl.pallas_call(
        _vq_body,
        grid=(N_TILES,),
        in_specs=[
            pl.BlockSpec((TILE, LATENT_DIM), lambda t: (t, 0)),
            pl.BlockSpec((SIZE, LATENT_DIM), lambda t: (0, 0)),
        ],
        out_specs=[
            pl.BlockSpec((TILE, LATENT_DIM), lambda t: (t, 0)),
            pl.BlockSpec((TILE, SIZE), lambda t: (t, 0)),
            pl.BlockSpec((1, 1, TILE), lambda t: (t, 0, 0)),
            pl.BlockSpec((1, 1), lambda t: (0, 0), memory_space=pltpu.SMEM),
            pl.BlockSpec((1, 1), lambda t: (0, 0), memory_space=pltpu.SMEM),
        ],
        out_shape=[
            jax.ShapeDtypeStruct((N_ROWS, LATENT_DIM), jnp.float32),
            jax.ShapeDtypeStruct((N_ROWS, SIZE), jnp.float32),
            jax.ShapeDtypeStruct((N_TILES, 1, TILE), jnp.int32),
            jax.ShapeDtypeStruct((1, 1), jnp.float32),
            jax.ShapeDtypeStruct((1, 1), jnp.float32),
        ],
        scratch_shapes=[
            pltpu.VMEM((1, SIZE), jnp.float32),
            pltpu.SMEM((1,), jnp.float32),
            pltpu.VMEM((1, SIZE), jnp.float32),
        ],
    )(zp, embedding_weight)

    z_q_out = jnp.transpose(zq.reshape(4, 8, 32, 32, LATENT_DIM), (0, 4, 1, 2, 3))
    min_idx = idx3.reshape(N_ROWS, 1)
    return (z_q_out, loss[0, 0], perp[0, 0], enc, min_idx)


# speedup vs baseline: 3.1560x; 1.1579x over previous
import jax
import jax.numpy as jnp
from jax.experimental import pallas as pl
from jax.experimental.pallas import tpu as pltpu

SIZE = 1024
LATENT_DIM = 256
BETA_C = 0.25
N_ROWS = 4 * 8 * 32 * 32
TILE = 1024
N_TILES = N_ROWS // TILE


def _vq_body(zr_ref, emb_ref, zq_ref, enc_ref, idx_ref, loss_ref, perp_ref,
             counts_ref, loss_acc_ref, en_ref):
    t = pl.program_id(0)

    e = emb_ref[...]

    @pl.when(t == 0)
    def _init():
        counts_ref[...] = jnp.zeros_like(counts_ref)
        loss_acc_ref[0] = 0.0
        en_ref[...] = jnp.sum(e * e, axis=1, keepdims=True)

    zr = zr_ref[...]
    en = en_ref[...]

    ezT = jax.lax.dot_general(e, zr, (((1,), (1,)), ((), ())),
                              preferred_element_type=jnp.float32)
    sT = en - 2.0 * ezT

    minv = jnp.min(sT, axis=0)
    code_iota_T = jax.lax.broadcasted_iota(jnp.int32, (SIZE, TILE), 0)
    idx = jnp.min(jnp.where(sT == minv[None, :], code_iota_T, SIZE), axis=0)

    code_iota = jax.lax.broadcasted_iota(jnp.int32, (TILE, SIZE), 1)
    oh = (code_iota == idx[:, None]).astype(jnp.float32)
    enc_ref[...] = oh
    zq = jax.lax.dot_general(oh, e, (((1,), (0,)), ((), ())),
                             preferred_element_type=jnp.float32)
    zq_ref[...] = zr + (zq - zr)

    counts_ref[...] += jnp.sum(oh, axis=0, keepdims=True)
    idx_ref[0, 0] = idx

    loss_acc_ref[0] += jnp.sum(zr * zr) + jnp.sum(minv)

    @pl.when(t == N_TILES - 1)
    def _finish():
        loss_ref[0, 0] = BETA_C * loss_acc_ref[0] / (N_ROWS * LATENT_DIM)
        e_mean = counts_ref[...] / N_ROWS
        perp_ref[0, 0] = jnp.exp(-jnp.sum(e_mean * jnp.log(e_mean + 1e-10)))


def kernel(z, embedding_weight):
    zp = jnp.transpose(z, (0, 2, 3, 4, 1)).reshape(N_ROWS, LATENT_DIM)
    zq, enc, idx3, loss, perp = pl.pallas_call(
        _vq_body,
        grid=(N_TILES,),
        in_specs=[
            pl.BlockSpec((TILE, LATENT_DIM), lambda t: (t, 0)),
            pl.BlockSpec((SIZE, LATENT_DIM), lambda t: (0, 0)),
        ],
        out_specs=[
            pl.BlockSpec((TILE, LATENT_DIM), lambda t: (t, 0)),
            pl.BlockSpec((TILE, SIZE), lambda t: (t, 0)),
            pl.BlockSpec((1, 1, TILE), lambda t: (t, 0, 0)),
            pl.BlockSpec((1, 1), lambda t: (0, 0), memory_space=pltpu.SMEM),
            pl.BlockSpec((1, 1), lambda t: (0, 0), memory_space=pltpu.SMEM),
        ],
        out_shape=[
            jax.ShapeDtypeStruct((N_ROWS, LATENT_DIM), jnp.float32),
            jax.ShapeDtypeStruct((N_ROWS, SIZE), jnp.float32),
            jax.ShapeDtypeStruct((N_TILES, 1, TILE), jnp.int32),
            jax.ShapeDtypeStruct((1, 1), jnp.float32),
            jax.ShapeDtypeStruct((1, 1), jnp.float32),
        ],
        scratch_shapes=[
            pltpu.VMEM((1, SIZE), jnp.float32),
            pltpu.SMEM((1,), jnp.float32),
            pltpu.VMEM((SIZE, 1), jnp.float32),
        ],
    )(zp, embedding_weight)

    z_q_out = jnp.transpose(zq.reshape(4, 8, 32, 32, LATENT_DIM), (0, 4, 1, 2, 3))
    min_idx = idx3.reshape(N_ROWS, 1)
    return (z_q_out, loss[0, 0], perp[0, 0], enc, min_idx)


# speedup vs baseline: 3.3062x; 1.0476x over previous
import jax
import jax.numpy as jnp
from jax.experimental import pallas as pl
from jax.experimental.pallas import tpu as pltpu

SIZE = 1024
LATENT_DIM = 256
BETA_C = 0.25
N_ROWS = 4 * 8 * 32 * 32
TILE = 1024
N_TILES = N_ROWS // TILE


def _vq_body(zr_ref, emb_ref, zq_ref, enc_ref, idx_ref, loss_ref, perp_ref,
             counts_ref, loss_acc_ref, en_ref, e2_ref):
    t = pl.program_id(0)

    e = emb_ref[...]

    @pl.when(t == 0)
    def _init():
        counts_ref[...] = jnp.zeros_like(counts_ref)
        loss_acc_ref[0] = 0.0
        en_ref[...] = jnp.sum(e * e, axis=1, keepdims=True)
        e2_ref[...] = -2.0 * e

    zr = zr_ref[...]
    en = en_ref[...]

    ezT = jax.lax.dot_general(e2_ref[...], zr, (((1,), (1,)), ((), ())),
                              preferred_element_type=jnp.float32)
    sT = en + ezT

    minv = jnp.min(sT, axis=0)
    code_iota_T = jax.lax.broadcasted_iota(jnp.int32, (SIZE, TILE), 0)
    idx = jnp.min(jnp.where(sT == minv[None, :], code_iota_T, SIZE), axis=0)

    code_iota = jax.lax.broadcasted_iota(jnp.int32, (TILE, SIZE), 1)
    oh = (code_iota == idx[:, None]).astype(jnp.float32)
    enc_ref[...] = oh
    zq = jax.lax.dot_general(oh, e, (((1,), (0,)), ((), ())),
                             preferred_element_type=jnp.float32)
    zq_ref[...] = zr + (zq - zr)

    ones_row = jnp.ones((1, TILE), jnp.float32)
    counts_ref[...] += jax.lax.dot_general(
        ones_row, oh, (((1,), (0,)), ((), ())),
        preferred_element_type=jnp.float32)
    idx_ref[0, 0] = idx

    loss_acc_ref[0] += jnp.sum(zr * zr) + jnp.sum(minv)

    @pl.when(t == N_TILES - 1)
    def _finish():
        loss_ref[0, 0] = BETA_C * loss_acc_ref[0] / (N_ROWS * LATENT_DIM)
        e_mean = counts_ref[...] / N_ROWS
        perp_ref[0, 0] = jnp.exp(-jnp.sum(e_mean * jnp.log(e_mean + 1e-10)))


def kernel(z, embedding_weight):
    zp = jnp.transpose(z, (0, 2, 3, 4, 1)).reshape(N_ROWS, LATENT_DIM)
    zq, enc, idx3, loss, perp = pl.pallas_call(
        _vq_body,
        grid=(N_TILES,),
        in_specs=[
            pl.BlockSpec((TILE, LATENT_DIM), lambda t: (t, 0)),
            pl.BlockSpec((SIZE, LATENT_DIM), lambda t: (0, 0)),
        ],
        out_specs=[
            pl.BlockSpec((TILE, LATENT_DIM), lambda t: (t, 0)),
            pl.BlockSpec((TILE, SIZE), lambda t: (t, 0)),
            pl.BlockSpec((1, 1, TILE), lambda t: (t, 0, 0)),
            pl.BlockSpec((1, 1), lambda t: (0, 0), memory_space=pltpu.SMEM),
            pl.BlockSpec((1, 1), lambda t: (0, 0), memory_space=pltpu.SMEM),
        ],
        out_shape=[
            jax.ShapeDtypeStruct((N_ROWS, LATENT_DIM), jnp.float32),
            jax.ShapeDtypeStruct((N_ROWS, SIZE), jnp.float32),
            jax.ShapeDtypeStruct((N_TILES, 1, TILE), jnp.int32),
            jax.ShapeDtypeStruct((1, 1), jnp.float32),
            jax.ShapeDtypeStruct((1, 1), jnp.float32),
        ],
        scratch_shapes=[
            pltpu.VMEM((1, SIZE), jnp.float32),
            pltpu.SMEM((1,), jnp.float32),
            pltpu.VMEM((SIZE, 1), jnp.float32),
            pltpu.VMEM((SIZE, LATENT_DIM), jnp.float32),
        ],
    )(zp, embedding_weight)

    z_q_out = jnp.transpose(zq.reshape(4, 8, 32, 32, LATENT_DIM), (0, 4, 1, 2, 3))
    min_idx = idx3.reshape(N_ROWS, 1)
    return (z_q_out, loss[0, 0], perp[0, 0], enc, min_idx)
